# baseline (speedup 1.0000x reference)
import jax
import jax.numpy as jnp
from jax import lax
from jax.experimental import pallas as pl
from jax.experimental.pallas import tpu as pltpu

_BLK = 200
_WAYS = 10
_GRID = 5


def _mlp_kernel(*refs):
    xs = refs[:_WAYS]
    w0t_ref, b0_ref, w1_ref, b1_ref, w2r_ref, b2_ref, out_ref, stage = \
        refs[_WAYS:]
    i = pl.program_id(0)

    w1 = w1_ref[...]
    w01t = lax.dot_general(w1, w0t_ref[...], (((0,), (0,)), ((), ())),
                           preferred_element_type=jnp.float32)
    b01t = lax.dot_general(w1, b0_ref[...], (((0,), (1,)), ((), ())),
                           preferred_element_type=jnp.float32) + b1_ref[...].T
    w2r = w2r_ref[...]

    outs = []
    for xr in xs:
        h_t = lax.dot_general(w01t, xr[...], (((1,), (1,)), ((), ())),
                              preferred_element_type=jnp.float32)
        h_t = jnp.maximum(h_t + b01t, 0.0)
        outs.append(lax.dot_general(w2r, h_t, (((1,), (0,)), ((), ())),
                                    preferred_element_type=jnp.float32))
    stage[i] = jnp.concatenate(outs, axis=1)

    @pl.when(i == _GRID - 1)
    def _():
        rows = [stage[k] for k in range(_GRID)]
        out_ref[...] = jnp.concatenate(rows, axis=1) + b2_ref[...]


def kernel(x, edge_index, W0, b0, W1, b1, W2, b2):
    del edge_index
    n, d = x.shape
    hid = W0.shape[1]
    end_hid = W1.shape[1]
    out_dim = W2.shape[1]
    x_specs = [
        pl.BlockSpec((_BLK, d), lambda i, j=j: (_WAYS * i + j, 0))
        for j in range(_WAYS)
    ]
    out = pl.pallas_call(
        _mlp_kernel,
        grid=(_GRID,),
        in_specs=x_specs + [
            pl.BlockSpec((hid, d), lambda i: (0, 0)),
            pl.BlockSpec((1, hid), lambda i: (0, 0)),
            pl.BlockSpec((hid, end_hid), lambda i: (0, 0)),
            pl.BlockSpec((1, end_hid), lambda i: (0, 0)),
            pl.BlockSpec((1, end_hid), lambda i: (0, 0)),
            pl.BlockSpec((1, out_dim), lambda i: (0, 0)),
        ],
        out_specs=pl.BlockSpec((1, n), lambda i: (0, 0)),
        out_shape=jax.ShapeDtypeStruct((1, n), jnp.float32),
        scratch_shapes=[pltpu.VMEM((_GRID, 1, _BLK * _WAYS), jnp.float32)],
        compiler_params=pltpu.CompilerParams(
            dimension_semantics=("arbitrary",)),
    )(x, x, x, x, x, x, x, x, x, x,
      W0.T, b0.reshape(1, hid), W1, b1.reshape(1, end_hid),
      W2.reshape(1, end_hid), b2.reshape(1, out_dim))
    return out.reshape(n, out_dim)


# speedup vs baseline: 1.7084x; 1.7084x over previous
import jax
import jax.numpy as jnp
from jax import lax
from jax.experimental import pallas as pl
from jax.experimental.pallas import tpu as pltpu

_BLK = 1000
_WAYS = 5
_GRID = 2


def _mlp_kernel(*refs):
    xs = refs[:_WAYS]
    w0t_ref, b0_ref, w1_ref, b1_ref, w2r_ref, b2_ref, out_ref, stage = \
        refs[_WAYS:]
    i = pl.program_id(0)

    w1 = w1_ref[...]
    w01t = lax.dot_general(w1, w0t_ref[...], (((0,), (0,)), ((), ())),
                           preferred_element_type=jnp.float32)
    b01t = lax.dot_general(w1, b0_ref[...], (((0,), (1,)), ((), ())),
                           preferred_element_type=jnp.float32) + b1_ref[...].T
    w2r = w2r_ref[...]

    outs = []
    for xr in xs:
        h_t = lax.dot_general(w01t, xr[...], (((1,), (1,)), ((), ())),
                              preferred_element_type=jnp.float32)
        h_t = jnp.maximum(h_t + b01t, 0.0)
        outs.append(lax.dot_general(w2r, h_t, (((1,), (0,)), ((), ())),
                                    preferred_element_type=jnp.float32))
    stage[i] = jnp.concatenate(outs, axis=1)

    @pl.when(i == _GRID - 1)
    def _():
        rows = [stage[k] for k in range(_GRID)]
        out_ref[...] = jnp.concatenate(rows, axis=1) + b2_ref[...]


def kernel(x, edge_index, W0, b0, W1, b1, W2, b2):
    del edge_index
    n, d = x.shape
    hid = W0.shape[1]
    end_hid = W1.shape[1]
    out_dim = W2.shape[1]
    x_specs = [
        pl.BlockSpec((_BLK, d), lambda i, j=j: (_WAYS * i + j, 0))
        for j in range(_WAYS)
    ]
    out = pl.pallas_call(
        _mlp_kernel,
        grid=(_GRID,),
        in_specs=x_specs + [
            pl.BlockSpec((hid, d), lambda i: (0, 0)),
            pl.BlockSpec((1, hid), lambda i: (0, 0)),
            pl.BlockSpec((hid, end_hid), lambda i: (0, 0)),
            pl.BlockSpec((1, end_hid), lambda i: (0, 0)),
            pl.BlockSpec((1, end_hid), lambda i: (0, 0)),
            pl.BlockSpec((1, out_dim), lambda i: (0, 0)),
        ],
        out_specs=pl.BlockSpec((1, n), lambda i: (0, 0)),
        out_shape=jax.ShapeDtypeStruct((1, n), jnp.float32),
        scratch_shapes=[pltpu.VMEM((_GRID, 1, _BLK * _WAYS), jnp.float32)],
        compiler_params=pltpu.CompilerParams(
            dimension_semantics=("arbitrary",)),
    )(x, x, x, x, x,
      W0.T, b0.reshape(1, hid), W1, b1.reshape(1, end_hid),
      W2.reshape(1, end_hid), b2.reshape(1, out_dim))
    return out.reshape(n, out_dim)
